# baseline (speedup 1.0000x reference)
import dataclasses

import jax
import jax.numpy as jnp
from jax.experimental import pallas as pl
from jax.experimental.pallas import tpu as pltpu
from jax.experimental.pallas import tpu_sc as plsc

_B = 16
_N = 32768
_NTILES = 32
_CH = _N // _NTILES
_NV = _CH // 16

_mesh = plsc.VectorSubcoreMesh(core_axis_name="c", subcore_axis_name="s")

_F = jnp.float32

_cp = pltpu.CompilerParams()
if "needs_layout_passes" in pltpu.CompilerParams.__dataclass_fields__:
    _cp = dataclasses.replace(_cp, needs_layout_passes=False)


def _vfull(v):
    return jnp.full((16,), v, dtype=_F)


def _tile_id():
    return jax.lax.axis_index("c") * 16 + jax.lax.axis_index("s")


@jax.named_call
def _k1_minmax(z, seg):
    @pl.kernel(
        out_type=jax.ShapeDtypeStruct((64, 16), _F),
        mesh=_mesh,
        compiler_params=_cp,
        scratch_types=[
            pltpu.VMEM((_CH,), _F),
            pltpu.VMEM((_CH,), jnp.int32),
            pltpu.VMEM((16,), _F),
            pltpu.VMEM((16,), _F),
        ],
    )
    def body(z_hbm, seg_hbm, out_hbm, z_t, seg_t, pmin_t, pmax_t):
        tile = _tile_id()
        c0 = tile * _CH
        pltpu.sync_copy(z_hbm.at[pl.ds(c0, _CH)], z_t)
        pltpu.sync_copy(seg_hbm.at[pl.ds(c0, _CH)], seg_t)
        smin = seg_t[pl.ds(0, 16)][0]
        smax = seg_t[pl.ds(_CH - 16, 16)][15]
        lane = jax.lax.iota(jnp.int32, 16)

        def seg_body(s, carry):
            rmin, rmax = carry

            def vec_body(i, c):
                vmin, vmax = c
                sl = pl.ds(i * 16, 16)
                zv = z_t[sl]
                m = seg_t[sl] == s
                vmin = jnp.minimum(vmin, jnp.where(m, zv, _vfull(jnp.inf)))
                vmax = jnp.maximum(vmax, jnp.where(m, zv, _vfull(-jnp.inf)))
                return vmin, vmax

            vmin, vmax = jax.lax.fori_loop(
                0, _NV, vec_body, (_vfull(jnp.inf), _vfull(-jnp.inf)))
            here = lane == s
            rmin = jnp.where(here, jnp.broadcast_to(jnp.min(vmin), (16,)), rmin)
            rmax = jnp.where(here, jnp.broadcast_to(jnp.max(vmax), (16,)), rmax)
            return rmin, rmax

        rmin, rmax = jax.lax.fori_loop(
            smin, smax + 1, seg_body, (_vfull(jnp.inf), _vfull(-jnp.inf)))
        pmin_t[...] = rmin
        pmax_t[...] = rmax
        pltpu.sync_copy(pmin_t, out_hbm.at[tile])
        pltpu.sync_copy(pmax_t, out_hbm.at[32 + tile])

    return body(z, seg)


@jax.named_call
def _k2_sums(planes, seg, mm):
    @pl.kernel(
        out_type=jax.ShapeDtypeStruct((128, 16), _F),
        mesh=_mesh,
        compiler_params=_cp,
        scratch_types=[
            pltpu.VMEM((_CH,), _F),
            pltpu.VMEM((_CH,), _F),
            pltpu.VMEM((_CH,), _F),
            pltpu.VMEM((_CH,), _F),
            pltpu.VMEM((_CH,), _F),
            pltpu.VMEM((_CH,), _F),
            pltpu.VMEM((_CH,), jnp.int32),
            pltpu.VMEM((3, _CH), _F),
            pltpu.VMEM((64, 16), _F),
            pltpu.VMEM((4, 16), _F),
            pltpu.VMEM((4, 16), _F),
        ],
    )
    def body(px_hbm, py_hbm, pz_hbm, tx_hbm, ty_hbm, tz_hbm, seg_hbm,
             mm_hbm, out_hbm,
             px_t, py_t, pz_t, tx_t, ty_t, tz_t, seg_t, val_t, mm_t,
             thr_t, acc_t):
        tile = _tile_id()
        c0 = tile * _CH
        sl_c = pl.ds(c0, _CH)
        pltpu.sync_copy(px_hbm.at[sl_c], px_t)
        pltpu.sync_copy(py_hbm.at[sl_c], py_t)
        pltpu.sync_copy(pz_hbm.at[sl_c], pz_t)
        pltpu.sync_copy(tx_hbm.at[sl_c], tx_t)
        pltpu.sync_copy(ty_hbm.at[sl_c], ty_t)
        pltpu.sync_copy(tz_hbm.at[sl_c], tz_t)
        pltpu.sync_copy(seg_hbm.at[sl_c], seg_t)
        pltpu.sync_copy(mm_hbm, mm_t)

        def red_body(k, carry):
            mn, mx = carry
            return (jnp.minimum(mn, mm_t[k]), jnp.maximum(mx, mm_t[32 + k]))

        mn, mx = jax.lax.fori_loop(0, 32, red_body,
                                   (_vfull(jnp.inf), _vfull(-jnp.inf)))
        rng = mx - mn
        zb = mn + 0.405 * rng
        zt = mx - 0.405 * rng
        mid = (mn + mx) / 2.0
        half = 0.19 * (zt - zb) / 2.0
        thr_t[0] = zb
        thr_t[1] = zt
        thr_t[2] = mid - half
        thr_t[3] = mid + half

        def vec_a(i, _):
            sl = pl.ds(i * 16, 16)
            dx = px_t[sl] - tx_t[sl]
            dy = py_t[sl] - ty_t[sl]
            dz = pz_t[sl] - tz_t[sl]
            zv = tz_t[sl]
            sv = seg_t[sl]

            def hub1(d):
                ad = jnp.abs(d)
                return jnp.where(ad < 0.5, 0.5 * d * d, 0.5 * (ad - 0.25))

            hub = hub1(dx) + hub1(dy) + hub1(dz)
            sq = dx * dx + dy * dy + dz * dz
            zbv = plsc.load_gather(thr_t.at[0], [sv])
            ztv = plsc.load_gather(thr_t.at[1], [sv])
            fbv = plsc.load_gather(thr_t.at[2], [sv])
            ftv = plsc.load_gather(thr_t.at[3], [sv])
            fil = (zv >= zbv) & (zv <= ztv) & (zv >= fbv) & (zv <= ftv)
            filf = jnp.where(fil, _vfull(1.0), _vfull(0.0))
            val_t[0, sl] = filf * hub
            val_t[1, sl] = (1.0 - filf) * sq
            val_t[2, sl] = filf
            return 0

        jax.lax.fori_loop(0, _NV, vec_a, 0)

        smin = seg_t[pl.ds(0, 16)][0]
        smax = seg_t[pl.ds(_CH - 16, 16)][15]
        lane = jax.lax.iota(jnp.int32, 16)

        def seg_body(s, carry):
            afs, afc, aes, acm = carry

            def vec_b(i, c):
                fs, fc, es, cm = c
                sl = pl.ds(i * 16, 16)
                m = seg_t[sl] == s
                z16 = _vfull(0.0)
                fs = fs + jnp.where(m, val_t[0, sl], z16)
                fc = fc + jnp.where(m, val_t[2, sl], z16)
                es = es + jnp.where(m, val_t[1, sl], z16)
                cm = cm + jnp.where(m, _vfull(1.0), z16)
                return fs, fc, es, cm

            z16 = _vfull(0.0)
            fs, fc, es, cm = jax.lax.fori_loop(0, _NV, vec_b,
                                               (z16, z16, z16, z16))
            here = lane == s
            afs = jnp.where(here, jnp.broadcast_to(jnp.sum(fs), (16,)), afs)
            afc = jnp.where(here, jnp.broadcast_to(jnp.sum(fc), (16,)), afc)
            aes = jnp.where(here, jnp.broadcast_to(jnp.sum(es), (16,)), aes)
            acm = jnp.where(here, jnp.broadcast_to(jnp.sum(cm), (16,)), acm)
            return afs, afc, aes, acm

        z16 = _vfull(0.0)
        afs, afc, aes, acm = jax.lax.fori_loop(
            smin, smax + 1, seg_body, (z16, z16, z16, z16))
        acc_t[0] = afs
        acc_t[1] = afc
        acc_t[2] = aes
        acc_t[3] = acm
        for q in range(4):
            pltpu.sync_copy(acc_t.at[q], out_hbm.at[32 * q + tile])

    return body(*planes, seg, mm)


@jax.named_call
def _k3_final(parts):
    @pl.kernel(
        out_type=jax.ShapeDtypeStruct((16,), _F),
        mesh=_mesh,
        compiler_params=_cp,
        scratch_types=[
            pltpu.VMEM((128, 16), _F),
            pltpu.VMEM((16,), _F),
        ],
    )
    def body(part_hbm, out_hbm, part_t, out_t):
        tile = _tile_id()

        def do_final(_):
            pltpu.sync_copy(part_hbm, part_t)

            def red(k, carry):
                fs, fc, es, cm = carry
                return (fs + part_t[k], fc + part_t[32 + k],
                        es + part_t[64 + k], cm + part_t[96 + k])

            z16 = _vfull(0.0)
            fs, fc, es, cm = jax.lax.fori_loop(0, 32, red, (z16, z16, z16, z16))
            ec = cm - fc
            fil_mean = jnp.where(fc > 0, fs / (3.0 * jnp.maximum(fc, 1.0)), z16)
            ele_mean = jnp.where(ec > 0, es / (3.0 * jnp.maximum(ec, 1.0)), z16)
            loss = (50.0 / _B) * jnp.sum(fil_mean) + (1.0 / _B) * jnp.sum(ele_mean)
            out_t[...] = jnp.broadcast_to(loss, (16,))
            pltpu.sync_copy(out_t, out_hbm)
            return 0

        jax.lax.cond(tile == 0, do_final, lambda _: 0, 0)

    return body(parts)


@jax.jit
def kernel(pred_coords, true_coords, batch_vector):
    planes = tuple(pred_coords[:, i] for i in range(3)) + tuple(
        true_coords[:, i] for i in range(3))
    seg = batch_vector.astype(jnp.int32)
    mm = _k1_minmax(planes[5], seg)
    parts = _k2_sums(planes, seg, mm)
    out = _k3_final(parts)
    return out[0]
